# baseline (speedup 1.0000x reference)
import jax
import jax.numpy as jnp
from jax import lax
from jax.experimental import pallas as pl
from jax.experimental.pallas import tpu as pltpu
import jax.experimental.pallas.tpu_sc as plsc

BATCH = 16384
D = 16
NC = 2
NS = 16
L = 16
NW = NC * NS
BPW = BATCH // NW
CHUNK = 128
NCHUNK = BPW // CHUNK


def _body(uid_hbm, iid_hbm, ut_hbm, it_hbm, w_hbm, b_hbm, out_hbm,
          idx_u, idx_i, u_c0, u_c1, i_c0, i_c1, out_v, wv, bv,
          sem0, sem1):
  c = lax.axis_index("c")
  s = lax.axis_index("s")
  wid = s * NC + c
  base = wid * BPW

  pltpu.sync_copy(uid_hbm.at[pl.ds(base, BPW)], idx_u)
  pltpu.sync_copy(iid_hbm.at[pl.ds(base, BPW)], idx_i)
  pltpu.sync_copy(w_hbm, wv)
  pltpu.sync_copy(b_hbm, bv)

  u_bufs = (u_c0, u_c1)
  i_bufs = (i_c0, i_c1)
  sems = (sem0, sem1)

  def fire(j):
    sl = pl.ds(j * CHUNK, CHUNK)
    cps = []
    for d in range(D):
      cps.append(pltpu.async_copy(
          ut_hbm.at[d].at[idx_u.at[sl]], u_bufs[j % 2].at[d], sems[j % 2]))
      cps.append(pltpu.async_copy(
          it_hbm.at[d].at[idx_i.at[sl]], i_bufs[j % 2].at[d], sems[j % 2]))
    return cps

  w_s = wv[...]
  b_s = bv[...]

  cps = fire(0)
  for j in range(NCHUNK):
    nxt = fire(j + 1) if j + 1 < NCHUNK else None
    for cp in cps:
      cp.wait()
    u_c = u_bufs[j % 2]
    i_c = i_bufs[j % 2]

    def group(g, _):
      sl = pl.ds(g * L, L)
      acc = None
      for d in range(D):
        prod = u_c[d, sl] * i_c[d, sl]
        acc = prod if acc is None else acc + prod
      out_v[pl.ds(j * CHUNK + g * L, L)] = acc * w_s + b_s
      return 0

    lax.fori_loop(0, CHUNK // L, group, 0)
    cps = nxt

  pltpu.sync_copy(out_v, out_hbm.at[pl.ds(base, BPW)])


@jax.jit
def _run(userID, ItemID, user_table_t, item_table_t, w, b):
  mesh = plsc.VectorSubcoreMesh(core_axis_name="c", subcore_axis_name="s")
  f = pl.kernel(
      _body,
      out_type=jax.ShapeDtypeStruct((BATCH,), jnp.float32),
      mesh=mesh,
      scratch_types=[
          pltpu.VMEM((BPW,), jnp.int32),
          pltpu.VMEM((BPW,), jnp.int32),
          pltpu.VMEM((D, CHUNK), jnp.float32),
          pltpu.VMEM((D, CHUNK), jnp.float32),
          pltpu.VMEM((D, CHUNK), jnp.float32),
          pltpu.VMEM((D, CHUNK), jnp.float32),
          pltpu.VMEM((BPW,), jnp.float32),
          pltpu.VMEM((L,), jnp.float32),
          pltpu.VMEM((L,), jnp.float32),
          pltpu.SemaphoreType.DMA,
          pltpu.SemaphoreType.DMA,
      ],
      compiler_params=pltpu.CompilerParams(use_tc_tiling_on_sc=False),
  )
  return f(userID, ItemID, user_table_t, item_table_t, w, b)


def kernel(userID, ItemID, user_table, item_table, w, b):
  w16 = jnp.broadcast_to(jnp.reshape(w, (1,)), (L,))
  b16 = jnp.broadcast_to(jnp.reshape(b, (1,)), (L,))
  return _run(userID.astype(jnp.int32), ItemID.astype(jnp.int32),
              user_table.T, item_table.T, w16, b16)


# speedup vs baseline: 19.2196x; 19.2196x over previous
import jax
import jax.numpy as jnp
from jax import lax
from jax.experimental import pallas as pl
from jax.experimental.pallas import tpu as pltpu
import jax.experimental.pallas.tpu_sc as plsc

BATCH = 16384
D = 16
NC = 2
NS = 16
L = 16
NW = NC * NS
BPW = BATCH // NW
GRP = 8
NGRP = BPW // GRP
IDX_PAD = BPW + L


def _body(uid_hbm, iid_hbm, ut_hbm, it_hbm, w_hbm, b_hbm, out_hbm,
          idx_u, idx_i, u_a, u_b, i_a, i_b, out_v, wv, bv,
          sem_a, sem_b):
  c = lax.axis_index("c")
  s = lax.axis_index("s")
  wid = s * NC + c
  base = wid * BPW

  pltpu.sync_copy(uid_hbm.at[pl.ds(base, BPW)], idx_u.at[pl.ds(0, BPW)])
  pltpu.sync_copy(iid_hbm.at[pl.ds(base, BPW)], idx_i.at[pl.ds(0, BPW)])
  idx_u[pl.ds(BPW, L)] = jnp.zeros((L,), jnp.int32)
  idx_i[pl.ds(BPW, L)] = jnp.zeros((L,), jnp.int32)
  pltpu.sync_copy(w_hbm, wv)
  pltpu.sync_copy(b_hbm, bv)

  u_bufs = (u_a, u_b)
  i_bufs = (i_a, i_b)
  sems = (sem_a, sem_b)
  w_s = wv[...]
  b_s = bv[...]
  iota = lax.iota(jnp.int32, L)
  base_rows = (iota & (GRP - 1)) * D

  def fire(g, par):
    v_u = idx_u[pl.ds(g * GRP, L)]
    v_i = idx_i[pl.ds(g * GRP, L)]
    for k in range(GRP):
      rb_u = pl.multiple_of((v_u[k] >> 7) * 128, 128)
      rb_i = pl.multiple_of((v_i[k] >> 7) * 128, 128)
      dst = pl.ds(k * D, D)
      pltpu.async_copy(ut_hbm.at[:, pl.ds(rb_u, 128)],
                       u_bufs[par].at[dst, :], sems[par])
      pltpu.async_copy(it_hbm.at[:, pl.ds(rb_i, 128)],
                       i_bufs[par].at[dst, :], sems[par])

  def drain(par):
    for k in range(GRP):
      dst = pl.ds(k * D, D)
      pltpu.make_async_copy(ut_hbm.at[:, pl.ds(0, 128)],
                            u_bufs[par].at[dst, :], sems[par]).wait()
      pltpu.make_async_copy(it_hbm.at[:, pl.ds(0, 128)],
                            i_bufs[par].at[dst, :], sems[par]).wait()

  def compute(q, par):
    v_u = idx_u[pl.ds(q * L, L)]
    v_i = idx_i[pl.ds(q * L, L)]
    cu = v_u & 127
    ci = v_i & 127
    u_c = u_bufs[par]
    i_c = i_bufs[par]
    acc = None
    for d in range(D):
      rows = base_rows + d
      ud = plsc.load_gather(u_c, [rows, cu])
      vd = plsc.load_gather(i_c, [rows, ci])
      prod = ud * vd
      acc = prod if acc is None else acc + prod
    return acc

  fire(0, 0)

  def pair(q, _):
    fire(2 * q + 1, 1)
    drain(0)
    acc_e = compute(q, 0)

    @pl.when(q < NGRP // 2 - 1)
    def _():
      fire(2 * q + 2, 0)

    drain(1)
    acc_o = compute(q, 1)
    res = jnp.where(iota < GRP, acc_e, acc_o)
    out_v[pl.ds(q * L, L)] = res * w_s + b_s
    return 0

  lax.fori_loop(0, NGRP // 2, pair, 0)

  pltpu.sync_copy(out_v, out_hbm.at[pl.ds(base, BPW)])


@jax.jit
def _run(userID, ItemID, user_table_t, item_table_t, w, b):
  mesh = plsc.VectorSubcoreMesh(core_axis_name="c", subcore_axis_name="s")
  f = pl.kernel(
      _body,
      out_type=jax.ShapeDtypeStruct((BATCH,), jnp.float32),
      mesh=mesh,
      scratch_types=[
          pltpu.VMEM((IDX_PAD,), jnp.int32),
          pltpu.VMEM((IDX_PAD,), jnp.int32),
          pltpu.VMEM((GRP * D, 128), jnp.float32),
          pltpu.VMEM((GRP * D, 128), jnp.float32),
          pltpu.VMEM((GRP * D, 128), jnp.float32),
          pltpu.VMEM((GRP * D, 128), jnp.float32),
          pltpu.VMEM((BPW,), jnp.float32),
          pltpu.VMEM((L,), jnp.float32),
          pltpu.VMEM((L,), jnp.float32),
          pltpu.SemaphoreType.DMA,
          pltpu.SemaphoreType.DMA,
      ],
      compiler_params=pltpu.CompilerParams(needs_layout_passes=False),
  )
  return f(userID, ItemID, user_table_t, item_table_t, w, b)


def kernel(userID, ItemID, user_table, item_table, w, b):
  w16 = jnp.broadcast_to(jnp.reshape(w, (1,)), (L,))
  b16 = jnp.broadcast_to(jnp.reshape(b, (1,)), (L,))
  return _run(userID.astype(jnp.int32), ItemID.astype(jnp.int32),
              user_table.T, item_table.T, w16, b16)
